# baseline (speedup 1.0000x reference)
import functools

import jax
import jax.numpy as jnp
from jax import lax
from jax.experimental import pallas as pl
from jax.experimental.pallas import tpu as pltpu
from jax.experimental.pallas import tpu_sc as plsc

_K = 1024
_D = 256
_B = 8
_N = 576
_BETA = 0.1
_ROWS = _B * _N
_BR = 512
_NBLK = _ROWS // _BR


def _vq_tc_body(zb_ref, et_ref, a_ref, c_ref, codes_ref, lacc_ref):
    i = pl.program_id(0)
    mm = lax.dot_general(
        zb_ref[...], et_ref[...],
        dimension_numbers=(((1,), (0,)), ((), ())),
        preferred_element_type=jnp.float32,
    )
    dist = a_ref[...] - 2.0 * mm + c_ref[...]
    minv = jnp.min(dist, axis=1)
    iota = lax.broadcasted_iota(jnp.int32, (_BR, _K), 1)
    code = jnp.min(jnp.where(dist == minv[:, None], iota, _K), axis=1)
    codes_ref[0, 0, :] = code

    @pl.when(i == 0)
    def _():
        lacc_ref[...] = jnp.zeros((1, 1), jnp.float32)

    lacc_ref[...] += jnp.sum(minv).reshape(1, 1)


def _tc_call(flat_z, et, a, c):
    return pl.pallas_call(
        _vq_tc_body,
        grid=(_NBLK,),
        in_specs=[
            pl.BlockSpec((_BR, _D), lambda i: (i, 0)),
            pl.BlockSpec((_D, _K), lambda i: (0, 0)),
            pl.BlockSpec((_BR, 1), lambda i: (i, 0)),
            pl.BlockSpec((1, _K), lambda i: (0, 0)),
        ],
        out_specs=[
            pl.BlockSpec((1, 1, _BR), lambda i: (i, 0, 0)),
            pl.BlockSpec((1, 1), lambda i: (0, 0)),
        ],
        out_shape=[
            jax.ShapeDtypeStruct((_NBLK, 1, _BR), jnp.int32),
            jax.ShapeDtypeStruct((1, 1), jnp.float32),
        ],
    )(flat_z, et, a, c)


def _make_sc_gather():
    info = plsc.get_sparse_core_info()
    nc, ns = info.num_cores, info.num_subcores
    nw = nc * ns
    bpw = _ROWS // nw
    mesh = plsc.VectorSubcoreMesh(core_axis_name="c", subcore_axis_name="s")

    @functools.partial(
        pl.kernel, mesh=mesh,
        out_type=jax.ShapeDtypeStruct((_ROWS, _D), jnp.float32),
        scratch_types=[
            pltpu.VMEM((bpw,), jnp.int32),
            pltpu.VMEM((bpw, _D), jnp.float32),
            pltpu.SemaphoreType.DMA,
        ],
    )
    def gather_k(codes_hbm, table_hbm, out_hbm, idx_v, rows_v, sem):
        wid = lax.axis_index("s") * nc + lax.axis_index("c")
        base = wid * bpw
        pltpu.sync_copy(codes_hbm.at[pl.ds(base, bpw)], idx_v)
        pltpu.async_copy(table_hbm.at[idx_v], rows_v, sem).wait()
        pltpu.sync_copy(rows_v, out_hbm.at[pl.ds(base, bpw)])

    return gather_k


def kernel(z, embed_weight):
    b, n, d = z.shape
    flat_z = z.reshape(-1, d)
    a = jnp.sum(flat_z ** 2, axis=1, keepdims=True)
    c = jnp.sum(embed_weight ** 2, axis=1).reshape(1, -1)
    et = embed_weight.T

    codes3d, lacc = _tc_call(flat_z, et, a, c)
    codes = codes3d.reshape(-1)

    z_q = _make_sc_gather()(codes, embed_weight)

    loss = _BETA * (lacc[0, 0] / (_ROWS * _D))
    return z_q.reshape(b, n, d), codes.reshape(b, n), loss


# speedup vs baseline: 1.5379x; 1.5379x over previous
import jax
import jax.numpy as jnp
from jax import lax
from jax.experimental import pallas as pl

_K = 1024
_D = 256
_B = 8
_N = 576
_BETA = 0.1
_ROWS = _B * _N
_BR = 512
_NBLK = _ROWS // _BR


def _vq_tc_body(zb_ref, et_ref, e_ref, a_ref, c_ref, codes_ref, zq_ref, lacc_ref):
    i = pl.program_id(0)
    mm = lax.dot_general(
        zb_ref[...], et_ref[...],
        dimension_numbers=(((1,), (0,)), ((), ())),
        preferred_element_type=jnp.float32,
    )
    dist = a_ref[...] - 2.0 * mm + c_ref[...]
    minv = jnp.min(dist, axis=1)
    iota_f = lax.broadcasted_iota(jnp.int32, (_BR, _K), 1).astype(jnp.float32)
    code_f = jnp.min(jnp.where(dist == minv[:, None], iota_f, float(_K)), axis=1)
    code = code_f.astype(jnp.int32)
    codes_ref[0, 0, :] = code
    onehot = jnp.where(iota_f == code_f[:, None], 1.0, 0.0)
    zq_ref[...] = lax.dot_general(
        onehot, e_ref[...],
        dimension_numbers=(((1,), (0,)), ((), ())),
        preferred_element_type=jnp.float32,
    )

    @pl.when(i == 0)
    def _():
        lacc_ref[...] = jnp.zeros((1, 1), jnp.float32)

    lacc_ref[...] += jnp.sum(minv).reshape(1, 1)


def _tc_call(flat_z, et, e, a, c):
    return pl.pallas_call(
        _vq_tc_body,
        grid=(_NBLK,),
        in_specs=[
            pl.BlockSpec((_BR, _D), lambda i: (i, 0)),
            pl.BlockSpec((_D, _K), lambda i: (0, 0)),
            pl.BlockSpec((_K, _D), lambda i: (0, 0)),
            pl.BlockSpec((_BR, 1), lambda i: (i, 0)),
            pl.BlockSpec((1, _K), lambda i: (0, 0)),
        ],
        out_specs=[
            pl.BlockSpec((1, 1, _BR), lambda i: (i, 0, 0)),
            pl.BlockSpec((_BR, _D), lambda i: (i, 0)),
            pl.BlockSpec((1, 1), lambda i: (0, 0)),
        ],
        out_shape=[
            jax.ShapeDtypeStruct((_NBLK, 1, _BR), jnp.int32),
            jax.ShapeDtypeStruct((_ROWS, _D), jnp.float32),
            jax.ShapeDtypeStruct((1, 1), jnp.float32),
        ],
    )(flat_z, et, e, a, c)


def kernel(z, embed_weight):
    b, n, d = z.shape
    flat_z = z.reshape(-1, d)
    a = jnp.sum(flat_z ** 2, axis=1, keepdims=True)
    c = jnp.sum(embed_weight ** 2, axis=1).reshape(1, -1)
    et = embed_weight.T

    codes3d, z_q, lacc = _tc_call(flat_z, et, embed_weight, a, c)
    codes = codes3d.reshape(-1)

    loss = _BETA * (lacc[0, 0] / (_ROWS * _D))
    return z_q.reshape(b, n, d), codes.reshape(b, n), loss


# speedup vs baseline: 1.5750x; 1.0241x over previous
import jax
import jax.numpy as jnp
from jax import lax
from jax.experimental import pallas as pl

_K = 1024
_D = 256
_B = 8
_N = 576
_BETA = 0.1
_ROWS = _B * _N
_BR = 512
_NBLK = _ROWS // _BR


def _vq_tc_body(zb_ref, e_ref, a_ref, c_ref, codes_ref, zq_ref, lacc_ref):
    i = pl.program_id(0)
    mm = lax.dot_general(
        zb_ref[...], e_ref[...],
        dimension_numbers=(((1,), (1,)), ((), ())),
        preferred_element_type=jnp.float32,
    )
    dist = a_ref[...] - 2.0 * mm + c_ref[...]
    minv = jnp.min(dist, axis=1)
    iota_f = lax.broadcasted_iota(jnp.int32, (_BR, _K), 1).astype(jnp.float32)
    code_f = jnp.min(jnp.where(dist == minv[:, None], iota_f, float(_K)), axis=1)
    code = code_f.astype(jnp.int32)
    codes_ref[0, 0, :] = code
    onehot = jnp.where(iota_f == code_f[:, None], 1.0, 0.0)
    zq_ref[...] = lax.dot_general(
        onehot, e_ref[...],
        dimension_numbers=(((1,), (0,)), ((), ())),
        preferred_element_type=jnp.float32,
    )

    @pl.when(i == 0)
    def _():
        lacc_ref[...] = jnp.zeros((1, 1), jnp.float32)

    lacc_ref[...] += jnp.sum(minv).reshape(1, 1)


def _tc_call(flat_z, e, a, c):
    return pl.pallas_call(
        _vq_tc_body,
        grid=(_NBLK,),
        in_specs=[
            pl.BlockSpec((_BR, _D), lambda i: (i, 0)),
            pl.BlockSpec((_K, _D), lambda i: (0, 0)),
            pl.BlockSpec((_BR, 1), lambda i: (i, 0)),
            pl.BlockSpec((1, _K), lambda i: (0, 0)),
        ],
        out_specs=[
            pl.BlockSpec((1, 1, _BR), lambda i: (i, 0, 0)),
            pl.BlockSpec((_BR, _D), lambda i: (i, 0)),
            pl.BlockSpec((1, 1), lambda i: (0, 0)),
        ],
        out_shape=[
            jax.ShapeDtypeStruct((_NBLK, 1, _BR), jnp.int32),
            jax.ShapeDtypeStruct((_ROWS, _D), jnp.float32),
            jax.ShapeDtypeStruct((1, 1), jnp.float32),
        ],
    )(flat_z, e, a, c)


def kernel(z, embed_weight):
    b, n, d = z.shape
    flat_z = z.reshape(-1, d)
    a = jnp.sum(flat_z ** 2, axis=1, keepdims=True)
    c = jnp.sum(embed_weight ** 2, axis=1).reshape(1, -1)

    codes3d, z_q, lacc = _tc_call(flat_z, embed_weight, a, c)
    codes = codes3d.reshape(-1)

    loss = _BETA * (lacc[0, 0] / (_ROWS * _D))
    return z_q.reshape(b, n, d), codes.reshape(b, n), loss


# speedup vs baseline: 1.9611x; 1.2452x over previous
import jax
import jax.numpy as jnp
from jax import lax
from jax.experimental import pallas as pl

_K = 1024
_D = 256
_B = 8
_N = 576
_BETA = 0.1
_ROWS = _B * _N
_BR = 512
_NBLK = _ROWS // _BR


def _vq_tc_body(zb_ref, e_ref, c_ref, codes_ref, zq_ref, lacc_ref):
    i = pl.program_id(0)
    zb = zb_ref[...]
    mm = lax.dot_general(
        zb, e_ref[...],
        dimension_numbers=(((1,), (1,)), ((), ())),
        preferred_element_type=jnp.float32,
    )
    a = jnp.sum(zb * zb, axis=1, keepdims=True)
    dist = a - 2.0 * mm + c_ref[...]
    minv = jnp.min(dist, axis=1)
    iota_f = lax.broadcasted_iota(jnp.int32, (_BR, _K), 1).astype(jnp.float32)
    code_f = jnp.min(jnp.where(dist == minv[:, None], iota_f, float(_K)), axis=1)
    code = code_f.astype(jnp.int32)
    codes_ref[0, 0, :] = code
    onehot = jnp.where(iota_f == code_f[:, None], 1.0, 0.0)
    zq_ref[...] = lax.dot_general(
        onehot, e_ref[...],
        dimension_numbers=(((1,), (0,)), ((), ())),
        preferred_element_type=jnp.float32,
    )

    @pl.when(i == 0)
    def _():
        lacc_ref[...] = jnp.zeros((1, 1), jnp.float32)

    lacc_ref[...] += jnp.sum(minv).reshape(1, 1)


def _tc_call(flat_z, e, c):
    return pl.pallas_call(
        _vq_tc_body,
        grid=(_NBLK,),
        in_specs=[
            pl.BlockSpec((_BR, _D), lambda i: (i, 0)),
            pl.BlockSpec((_K, _D), lambda i: (0, 0)),
            pl.BlockSpec((1, _K), lambda i: (0, 0)),
        ],
        out_specs=[
            pl.BlockSpec((1, 1, _BR), lambda i: (i, 0, 0)),
            pl.BlockSpec((_BR, _D), lambda i: (i, 0)),
            pl.BlockSpec((1, 1), lambda i: (0, 0)),
        ],
        out_shape=[
            jax.ShapeDtypeStruct((_NBLK, 1, _BR), jnp.int32),
            jax.ShapeDtypeStruct((_ROWS, _D), jnp.float32),
            jax.ShapeDtypeStruct((1, 1), jnp.float32),
        ],
    )(flat_z, e, c)


def kernel(z, embed_weight):
    b, n, d = z.shape
    flat_z = z.reshape(-1, d)
    c = jnp.sum(embed_weight ** 2, axis=1).reshape(1, -1)

    codes3d, z_q, lacc = _tc_call(flat_z, embed_weight, c)
    codes = codes3d.reshape(-1)

    loss = _BETA * (lacc[0, 0] / (_ROWS * _D))
    return z_q.reshape(b, n, d), codes.reshape(b, n), loss


# speedup vs baseline: 2.3229x; 1.1845x over previous
import jax
import jax.numpy as jnp
from jax import lax
from jax.experimental import pallas as pl
from jax.experimental.pallas import tpu as pltpu

_K = 1024
_D = 256
_B = 8
_N = 576
_BETA = 0.1
_ROWS = _B * _N
_BR = 512
_NBLK = _ROWS // _BR
_LSCALE = _BETA / (_ROWS * _D)


def _vq_tc_body(zb_ref, e_ref, codes_ref, zq_ref, lacc_ref, c_ref):
    i = pl.program_id(0)

    @pl.when(i == 0)
    def _():
        e = e_ref[...]
        c_ref[...] = jnp.sum(e * e, axis=1).reshape(1, _K)
        lacc_ref[...] = jnp.zeros((1, 1), jnp.float32)

    zb = zb_ref[...]
    zb2 = zb * -2.0
    mm2 = lax.dot_general(
        zb2, e_ref[...],
        dimension_numbers=(((1,), (1,)), ((), ())),
        preferred_element_type=jnp.float32,
    )
    a = jnp.sum(zb2 * zb2, axis=1, keepdims=True) * 0.25
    dist = a + mm2 + c_ref[...]
    minv = jnp.min(dist, axis=1)
    iota_f = lax.broadcasted_iota(jnp.int32, (_BR, _K), 1).astype(jnp.float32)
    sel = jnp.where(dist == minv[:, None], iota_f, float(_K))
    code_f = jnp.min(sel, axis=1)
    codes_ref[0, 0, :] = code_f.astype(jnp.int32)
    onehot = jnp.where(sel == code_f[:, None], 1.0, 0.0)
    zq_ref[...] = lax.dot_general(
        onehot, e_ref[...],
        dimension_numbers=(((1,), (0,)), ((), ())),
        preferred_element_type=jnp.float32,
    )

    lacc_ref[...] += jnp.sum(minv).reshape(1, 1)

    @pl.when(i == _NBLK - 1)
    def _():
        lacc_ref[...] = lacc_ref[...] * _LSCALE


def _tc_call(flat_z, e):
    return pl.pallas_call(
        _vq_tc_body,
        grid=(_NBLK,),
        in_specs=[
            pl.BlockSpec((_BR, _D), lambda i: (i, 0)),
            pl.BlockSpec((_K, _D), lambda i: (0, 0)),
        ],
        out_specs=[
            pl.BlockSpec((1, 1, _BR), lambda i: (i, 0, 0)),
            pl.BlockSpec((_BR, _D), lambda i: (i, 0)),
            pl.BlockSpec((1, 1), lambda i: (0, 0)),
        ],
        out_shape=[
            jax.ShapeDtypeStruct((_NBLK, 1, _BR), jnp.int32),
            jax.ShapeDtypeStruct((_ROWS, _D), jnp.float32),
            jax.ShapeDtypeStruct((1, 1), jnp.float32),
        ],
        scratch_shapes=[pltpu.VMEM((1, _K), jnp.float32)],
    )(flat_z, e)


def kernel(z, embed_weight):
    b, n, d = z.shape
    flat_z = z.reshape(-1, d)
    codes3d, z_q, lacc = _tc_call(flat_z, embed_weight)
    return z_q.reshape(b, n, d), codes3d.reshape(b, n), lacc[0, 0]
